# baseline (speedup 1.0000x reference)
import functools

import jax
import jax.numpy as jnp
from jax import lax
from jax.experimental import pallas as pl
from jax.experimental.pallas import tpu as pltpu
from jax.experimental.pallas import tpu_sc as plsc

VOCAB, D, H, C = 1000000, 32, 32, 10
B, L = 4096, 50

NUM_CORES = 2
NUM_SUBCORES = 16
NW = NUM_CORES * NUM_SUBCORES
SPW = B // NW
LP = 64
SR = 128
K = 4
NR = SPW // K
OB = B * D // SR

VB = 2048

_mesh = plsc.VectorSubcoreMesh(core_axis_name="c", subcore_axis_name="s")


def _detile_body(t_ref, o_ref):
    z = jnp.transpose(t_ref[...])
    z3 = jnp.reshape(z, (VB // 4, 4, D))
    o_ref[...] = jnp.concatenate([z3[:, q, :] for q in range(4)], axis=-1)


def _detile(tableT):
    grid = (VOCAB + VB - 1) // VB
    return pl.pallas_call(
        _detile_body,
        grid=(grid,),
        in_specs=[pl.BlockSpec((H, VB), lambda i: (0, i))],
        out_specs=pl.BlockSpec((VB // 4, SR), lambda i: (i, 0)),
        out_shape=jax.ShapeDtypeStruct((VOCAB * D // SR, SR), jnp.float32),
    )(tableT)


@functools.partial(
    pl.kernel,
    mesh=_mesh,
    compiler_params=pltpu.CompilerParams(use_tc_tiling_on_sc=True),
    out_type=jax.ShapeDtypeStruct((OB, SR), jnp.float32),
    scratch_types=[
        pltpu.VMEM((SPW, SR), jnp.int32),
        pltpu.VMEM((SPW, SR), jnp.int32),
        pltpu.VMEM((K, LP, SR), jnp.float32),
        pltpu.VMEM((K, LP, SR), jnp.float32),
        pltpu.VMEM((1, SR), jnp.float32),
        pltpu.VMEM((1, SR), jnp.float32),
        pltpu.SemaphoreType.DMA,
        pltpu.SemaphoreType.DMA,
        pltpu.SemaphoreType.DMA,
        pltpu.SemaphoreType.DMA,
    ],
)
def _sc_gather_sum(xsup_hbm, qoff_hbm, table_hbm, out_hbm,
                   idx_v, qoff_v, rows_a, rows_b, out_a, out_b,
                   sem_a, sem_b, sem_oa, sem_ob):
    wid = lax.axis_index("s") * NUM_CORES + lax.axis_index("c")
    base = wid * SPW
    pltpu.sync_copy(xsup_hbm.at[pl.ds(base, SPW)], idx_v)
    pltpu.sync_copy(qoff_hbm.at[pl.ds(base, SPW)], qoff_v)

    def issue(buf, sem, r):
        @pl.when(r < NR)
        def _():
            for j in range(K):
                s = r * K + j
                for c in range(LP // 16):
                    iv = idx_v[s, pl.ds(16 * c, 16)]
                    pltpu.async_copy(
                        table_hbm.at[iv],
                        buf.at[j, pl.ds(16 * c, 16)], sem)

    def drain(buf, sem):
        for j in range(K):
            for c in range(LP // 16):
                iv = idx_v[0, pl.ds(0, 16)]
                pltpu.make_async_copy(
                    table_hbm.at[iv],
                    buf.at[j, pl.ds(16 * c, 16)], sem).wait()

    def consume(buf, out_buf, r):
        for j in range(K):
            s = r * K + j
            qc = [qoff_v[s, pl.ds(16 * c, 16)] for c in range(4)]

            def qat(t):
                return qc[t // 16][t % 16]

            acc0 = buf[j, 0, pl.ds(qat(0), 16)]
            acc1 = buf[j, 0, pl.ds(qat(0) + 16, 16)]
            for t in range(1, L):
                q = qat(t)
                acc0 = acc0 + buf[j, t, pl.ds(q, 16)]
                acc1 = acc1 + buf[j, t, pl.ds(q + 16, 16)]
            out_buf[0, pl.ds(j * D, 16)] = acc0
            out_buf[0, pl.ds(j * D + 16, 16)] = acc1

    def store(out_buf, sem_o, r):
        pltpu.async_copy(
            out_buf, out_hbm.at[pl.ds(wid * NR + r, 1)], sem_o)

    def wait_store(out_buf, sem_o):
        pltpu.make_async_copy(
            out_buf, out_hbm.at[pl.ds(wid * NR, 1)], sem_o).wait()

    issue(rows_a, sem_a, 0)
    issue(rows_b, sem_b, 1)

    def body(g, _):
        ra = 2 * g
        rb = 2 * g + 1

        @pl.when(g > 0)
        def _():
            wait_store(out_a, sem_oa)
        drain(rows_a, sem_a)
        consume(rows_a, out_a, ra)
        issue(rows_a, sem_a, ra + 2)
        store(out_a, sem_oa, ra)

        @pl.when(g > 0)
        def _():
            wait_store(out_b, sem_ob)
        drain(rows_b, sem_b)
        consume(rows_b, out_b, rb)
        issue(rows_b, sem_b, rb + 2)
        store(out_b, sem_ob, rb)
        return 0

    lax.fori_loop(0, NR // 2, body, 0)
    wait_store(out_a, sem_oa)
    wait_store(out_b, sem_ob)


def _mlp_body(s_ref, l_ref, w1_ref, b1_ref, w2_ref, b2_ref, o_ref):
    rep = s_ref[...] * l_ref[...]
    h = lax.dot_general(rep, w1_ref[...], (((1,), (1,)), ((), ())),
                        preferred_element_type=jnp.float32) + b1_ref[...]
    h = jnp.maximum(h, 0.0)
    o_ref[...] = lax.dot_general(h, w2_ref[...], (((1,), (1,)), ((), ())),
                                 preferred_element_type=jnp.float32) + b2_ref[...]


@jax.jit
def kernel(x, lengths, table, W1, b1, W2, b2):
    xp = jnp.pad(x, ((0, 0), (0, SR - L)))
    xsup = xp >> 2
    qoff = (xp & 3) * D
    table2 = _detile(table.T)
    sums128 = _sc_gather_sum(xsup, qoff, table2)
    sums = sums128.reshape(B, D)
    inv_len = (1.0 / lengths.astype(jnp.float32)).reshape(B, 1)
    logits = pl.pallas_call(
        _mlp_body,
        out_shape=jax.ShapeDtypeStruct((B, C), jnp.float32),
    )(sums, inv_len, W1, b1.reshape(1, H), W2, b2.reshape(1, C))
    return logits


# speedup vs baseline: 1.0648x; 1.0648x over previous
import functools

import jax
import jax.numpy as jnp
from jax import lax
from jax.experimental import pallas as pl
from jax.experimental.pallas import tpu as pltpu
from jax.experimental.pallas import tpu_sc as plsc

VOCAB, D, H, C = 1000000, 32, 32, 10
B, L = 4096, 50

NUM_CORES = 2
NUM_SUBCORES = 16
NW = NUM_CORES * NUM_SUBCORES
SPW = B // NW
SR = 128
VPAD = 1000064
K = 4
NR = SPW // K
OB = B * D // SR

_mesh = plsc.VectorSubcoreMesh(core_axis_name="c", subcore_axis_name="s")


@functools.partial(
    pl.kernel,
    mesh=_mesh,
    compiler_params=pltpu.CompilerParams(use_tc_tiling_on_sc=False),
    out_type=jax.ShapeDtypeStruct((OB, SR), jnp.float32),
    scratch_types=[
        pltpu.VMEM((SPW, SR), jnp.int32),
        pltpu.VMEM((K, L * D), jnp.float32),
        pltpu.VMEM((K, L * D), jnp.float32),
        pltpu.VMEM((1, SR), jnp.float32),
        pltpu.VMEM((1, SR), jnp.float32),
        pltpu.SemaphoreType.DMA,
        pltpu.SemaphoreType.DMA,
        pltpu.SemaphoreType.DMA,
        pltpu.SemaphoreType.DMA,
    ],
)
def _sc_gather_sum(xp_hbm, table_hbm, out_hbm,
                   idx_v, rows_a, rows_b, out_a, out_b,
                   sem_a, sem_b, sem_oa, sem_ob):
    wid = lax.axis_index("s") * NUM_CORES + lax.axis_index("c")
    base = wid * SPW
    pltpu.sync_copy(xp_hbm.at[pl.ds(base, SPW)], idx_v)

    lane = lax.iota(jnp.int32, 16)
    dlo = lane * VPAD
    dhi = dlo + 16 * VPAD

    def issue(buf, sem, r):
        @pl.when(r < NR)
        def _():
            for j in range(K):
                s = r * K + j
                for c in range((L + 15) // 16):
                    vc = idx_v[s, pl.ds(16 * c, 16)]
                    for p in range(16):
                        t = 16 * c + p
                        if t >= L:
                            break
                        v = vc[p]
                        pltpu.async_copy(
                            table_hbm.at[dlo + v],
                            buf.at[j, pl.ds(t * D, 16)], sem)
                        pltpu.async_copy(
                            table_hbm.at[dhi + v],
                            buf.at[j, pl.ds(t * D + 16, 16)], sem)

    def drain(buf, sem):
        for j in range(K):
            pltpu.make_async_copy(
                table_hbm.at[pl.ds(0, L * D)], buf.at[j], sem).wait()

    def consume(buf, out_buf, r):
        for j in range(K):
            acc0 = buf[j, pl.ds(0, 16)]
            acc1 = buf[j, pl.ds(16, 16)]
            for t in range(1, L):
                acc0 = acc0 + buf[j, pl.ds(t * D, 16)]
                acc1 = acc1 + buf[j, pl.ds(t * D + 16, 16)]
            out_buf[0, pl.ds(j * D, 16)] = acc0
            out_buf[0, pl.ds(j * D + 16, 16)] = acc1

    def store(out_buf, sem_o, r):
        pltpu.async_copy(
            out_buf, out_hbm.at[pl.ds(wid * NR + r, 1)], sem_o)

    def wait_store(out_buf, sem_o):
        pltpu.make_async_copy(
            out_buf, out_hbm.at[pl.ds(wid * NR, 1)], sem_o).wait()

    issue(rows_a, sem_a, 0)
    issue(rows_b, sem_b, 1)

    def body(g, _):
        ra = 2 * g
        rb = 2 * g + 1

        @pl.when(g > 0)
        def _():
            wait_store(out_a, sem_oa)
        drain(rows_a, sem_a)
        consume(rows_a, out_a, ra)
        issue(rows_a, sem_a, ra + 2)
        store(out_a, sem_oa, ra)

        @pl.when(g > 0)
        def _():
            wait_store(out_b, sem_ob)
        drain(rows_b, sem_b)
        consume(rows_b, out_b, rb)
        issue(rows_b, sem_b, rb + 2)
        store(out_b, sem_ob, rb)
        return 0

    lax.fori_loop(0, NR // 2, body, 0)
    wait_store(out_a, sem_oa)
    wait_store(out_b, sem_ob)


def _mlp_body(s_ref, l_ref, w1_ref, b1_ref, w2_ref, b2_ref, o_ref):
    rep = s_ref[...] * l_ref[...]
    h = lax.dot_general(rep, w1_ref[...], (((1,), (1,)), ((), ())),
                        preferred_element_type=jnp.float32) + b1_ref[...]
    h = jnp.maximum(h, 0.0)
    o_ref[...] = lax.dot_general(h, w2_ref[...], (((1,), (1,)), ((), ())),
                                 preferred_element_type=jnp.float32) + b2_ref[...]


@jax.jit
def kernel(x, lengths, table, W1, b1, W2, b2):
    xp = jnp.pad(x, ((0, 0), (0, SR - L)))
    tablef = jnp.pad(table.T, ((0, 0), (0, VPAD - VOCAB))).reshape(H * VPAD)
    sums128 = _sc_gather_sum(xp, tablef)
    sums = sums128.reshape(B, D)
    inv_len = (1.0 / lengths.astype(jnp.float32)).reshape(B, 1)
    logits = pl.pallas_call(
        _mlp_body,
        out_shape=jax.ShapeDtypeStruct((B, C), jnp.float32),
    )(sums, inv_len, W1, b1.reshape(1, H), W2, b2.reshape(1, C))
    return logits


# speedup vs baseline: 3.5633x; 3.3465x over previous
import functools

import jax
import jax.numpy as jnp
from jax import lax
from jax.experimental import pallas as pl
from jax.experimental.pallas import tpu as pltpu
from jax.experimental.pallas import tpu_sc as plsc

VOCAB, D, H, C = 1000000, 32, 32, 10
B, L = 4096, 50

NUM_CORES = 2
NUM_SUBCORES = 16
NW = NUM_CORES * NUM_SUBCORES
LP = 56
PPW = B // (2 * NW)
LP2 = 2 * LP
K = 4
NR = PPW // K

_mesh = plsc.VectorSubcoreMesh(core_axis_name="c", subcore_axis_name="s")


def _tree_sum(vals):
    vals = list(vals)
    while len(vals) > 1:
        nxt = [vals[i] + vals[i + 1] for i in range(0, len(vals) - 1, 2)]
        if len(vals) % 2:
            nxt.append(vals[-1])
        vals = nxt
    return vals[0]


def _sum_sample(rows, j, row0, col):
    parts = []
    for bs in range(0, L, 8):
        grp = [rows[j, row0 + t, pl.ds(col * 16, 16)]
               for t in range(bs, min(bs + 8, L))]
        parts.append(_tree_sum(grp))
    return _tree_sum(parts)


@functools.partial(
    pl.kernel,
    mesh=_mesh,
    compiler_params=pltpu.CompilerParams(use_tc_tiling_on_sc=False),
    out_type=jax.ShapeDtypeStruct((B, D), jnp.float32),
    scratch_types=[
        pltpu.VMEM((PPW, LP2), jnp.int32),
        pltpu.VMEM((K, LP2, D), jnp.float32),
        pltpu.VMEM((K, LP2, D), jnp.float32),
        pltpu.VMEM((2 * K, D), jnp.float32),
        pltpu.VMEM((2 * K, D), jnp.float32),
        pltpu.SemaphoreType.DMA,
        pltpu.SemaphoreType.DMA,
        pltpu.SemaphoreType.DMA,
        pltpu.SemaphoreType.DMA,
    ],
)
def _sc_gather_sum(xp_hbm, table_hbm, out_hbm,
                   idx_v, rows_a, rows_b, out_a, out_b,
                   sem_a, sem_b, sem_oa, sem_ob):
    wid = lax.axis_index("s") * NUM_CORES + lax.axis_index("c")
    pbase = wid * PPW
    sbase = 2 * pbase
    pltpu.sync_copy(xp_hbm.at[pl.ds(pbase, PPW)], idx_v)

    def issue(buf, sem, r):
        @pl.when(r < NR)
        def _():
            for j in range(K):
                pltpu.async_copy(
                    table_hbm.at[idx_v.at[r * K + j]], buf.at[j], sem)

    def drain(buf, sem):
        for j in range(K):
            pltpu.make_async_copy(
                table_hbm.at[idx_v.at[0]], buf.at[j], sem).wait()

    def consume(buf, out_buf):
        for j in range(K):
            for h in range(2):
                out_buf[2 * j + h, pl.ds(0, 16)] = \
                    _sum_sample(buf, j, h * LP, 0)
                out_buf[2 * j + h, pl.ds(16, 16)] = \
                    _sum_sample(buf, j, h * LP, 1)

    def store(out_buf, sem_o, r):
        pltpu.async_copy(
            out_buf, out_hbm.at[pl.ds(sbase + r * 2 * K, 2 * K)], sem_o)

    def wait_store(out_buf, sem_o):
        pltpu.make_async_copy(
            out_buf, out_hbm.at[pl.ds(sbase, 2 * K)], sem_o).wait()

    issue(rows_a, sem_a, 0)
    issue(rows_b, sem_b, 1)

    def body(g, _):
        ra = 2 * g
        rb = 2 * g + 1

        @pl.when(g > 0)
        def _():
            wait_store(out_a, sem_oa)
        drain(rows_a, sem_a)
        consume(rows_a, out_a)
        issue(rows_a, sem_a, ra + 2)
        store(out_a, sem_oa, ra)

        @pl.when(g > 0)
        def _():
            wait_store(out_b, sem_ob)
        drain(rows_b, sem_b)
        consume(rows_b, out_b)
        issue(rows_b, sem_b, rb + 2)
        store(out_b, sem_ob, rb)
        return 0

    lax.fori_loop(0, NR // 2, body, 0)
    wait_store(out_a, sem_oa)
    wait_store(out_b, sem_ob)


def _mlp_body(s_ref, l_ref, w1_ref, b1_ref, w2_ref, b2_ref, o_ref):
    rep = s_ref[...] * l_ref[...]
    h = lax.dot_general(rep, w1_ref[...], (((1,), (1,)), ((), ())),
                        preferred_element_type=jnp.float32) + b1_ref[...]
    h = jnp.maximum(h, 0.0)
    o_ref[...] = lax.dot_general(h, w2_ref[...], (((1,), (1,)), ((), ())),
                                 preferred_element_type=jnp.float32) + b2_ref[...]


@jax.jit
def kernel(x, lengths, table, W1, b1, W2, b2):
    xp = jnp.pad(x, ((0, 0), (0, LP - L))).reshape(B // 2, LP2)
    sums = _sc_gather_sum(xp, table)
    inv_len = (1.0 / lengths.astype(jnp.float32)).reshape(B, 1)
    logits = pl.pallas_call(
        _mlp_body,
        out_shape=jax.ShapeDtypeStruct((B, C), jnp.float32),
    )(sums, inv_len, W1, b1.reshape(1, H), W2, b2.reshape(1, C))
    return logits
